# baseline (speedup 1.0000x reference)
import functools

import jax
import jax.numpy as jnp
import numpy as np
from jax import lax
from jax.experimental import pallas as pl
import jax.experimental.pallas.tpu as pltpu
from jax.experimental.pallas import tpu_sc as plsc

N = 10000
NPAD = 10240
E = 320000
D = 128
G = 64
TRASH = N
RB = 512
NRB = NPAD // RB
CR = 256

f32 = jnp.float32
i32 = jnp.int32


def _to_i32(v):
  v &= 0xFFFFFFFF
  return jnp.int32(v - 2**32 if v >= 2**31 else v)


def _mm_body(x_ref, w_ref, b_ref, o_ref):
  o_ref[...] = (
      jnp.dot(x_ref[...], w_ref[...], preferred_element_type=f32) + b_ref[...]
  )


def _k_mm(x, w, b):
  dout = w.shape[1]
  return pl.pallas_call(
      _mm_body,
      grid=(NRB,),
      in_specs=[
          pl.BlockSpec((RB, D), lambda i: (i, 0)),
          pl.BlockSpec((D, dout), lambda i: (0, 0)),
          pl.BlockSpec((1, dout), lambda i: (0, 0)),
      ],
      out_specs=pl.BlockSpec((RB, dout), lambda i: (i, 0)),
      out_shape=jax.ShapeDtypeStruct((NPAD, dout), f32),
  )(x, w, b)


def _fold_body(hpre_ref, da_ref, db_ref, ss_ref, dinv_ref, hp_ref):
  deg = da_ref[...] + db_ref[...] + ss_ref[...]
  dinv = lax.rsqrt(jnp.clip(deg, 1e-12, None))
  dinv_ref[...] = dinv
  hp_ref[...] = dinv * hpre_ref[...]


def _k_fold(hpre, dega, degb, selfsel):
  return pl.pallas_call(
      _fold_body,
      grid=(NRB,),
      in_specs=[
          pl.BlockSpec((RB, D), lambda i: (i, 0)),
          pl.BlockSpec((RB, 1), lambda i: (i, 0)),
          pl.BlockSpec((RB, 1), lambda i: (i, 0)),
          pl.BlockSpec((RB, 1), lambda i: (i, 0)),
      ],
      out_specs=[
          pl.BlockSpec((RB, 1), lambda i: (i, 0)),
          pl.BlockSpec((RB, D), lambda i: (i, 0)),
      ],
      out_shape=[
          jax.ShapeDtypeStruct((NPAD, 1), f32),
          jax.ShapeDtypeStruct((NPAD, D), f32),
      ],
  )(hpre, dega, degb, selfsel)


def _combine_body(sa_ref, sb_ref, hpre_ref, dinv_ref, ss_ref, wp_ref, bp_ref,
                  h_ref, pp_ref):
  dinv = dinv_ref[...]
  ss = ss_ref[...]
  h = jnp.maximum(
      dinv * (sa_ref[...] + sb_ref[...]) + dinv * dinv * ss * hpre_ref[...],
      0.0,
  )
  h_ref[...] = h
  pp_ref[...] = dinv * (
      jnp.dot(h, wp_ref[...], preferred_element_type=f32) + bp_ref[...]
  )


def _k_combine(sa, sb, hpre, dinv, selfsel, wp, bp):
  return pl.pallas_call(
      _combine_body,
      grid=(NRB,),
      in_specs=[
          pl.BlockSpec((RB, D), lambda i: (i, 0)),
          pl.BlockSpec((RB, D), lambda i: (i, 0)),
          pl.BlockSpec((RB, D), lambda i: (i, 0)),
          pl.BlockSpec((RB, 1), lambda i: (i, 0)),
          pl.BlockSpec((RB, 1), lambda i: (i, 0)),
          pl.BlockSpec((D, 1), lambda i: (0, 0)),
          pl.BlockSpec((1, 1), lambda i: (0, 0)),
      ],
      out_specs=[
          pl.BlockSpec((RB, D), lambda i: (i, 0)),
          pl.BlockSpec((RB, 1), lambda i: (i, 0)),
      ],
      out_shape=[
          jax.ShapeDtypeStruct((NPAD, D), f32),
          jax.ShapeDtypeStruct((NPAD, 1), f32),
      ],
  )(sa, sb, hpre, dinv, selfsel, wp, bp)


def _rpartials(mxin, bcol):
  g0 = bcol[0, 0]
  g1 = jnp.minimum(bcol[RB - 1, 0], G - 1)
  sumv = jnp.where(mxin > -jnp.inf, mxin, 0.0)
  giota = lax.broadcasted_iota(i32, (G, 1), 0)

  def body(g, carry):
    amx, asm = carry
    m = bcol == g
    mx = jnp.max(jnp.where(m, mxin, -jnp.inf), axis=0, keepdims=True)
    sm = jnp.sum(jnp.where(m, sumv, 0.0), axis=0, keepdims=True)
    amx = jnp.where(giota == g, mx, amx)
    asm = jnp.where(giota == g, sm, asm)
    return amx, asm

  amx0 = jnp.full((G, D), -jnp.inf, f32)
  asm0 = jnp.zeros((G, D), f32)
  return lax.fori_loop(g0, g1 + 1, body, (amx0, asm0))


def _combine3_body(sa_ref, sb_ref, hpre_ref, dinv_ref, ss_ref, batch_ref,
                   h_ref, pmx_ref, psm_ref):
  dinv = dinv_ref[...]
  ss = ss_ref[...]
  h = jnp.maximum(
      dinv * (sa_ref[...] + sb_ref[...]) + dinv * dinv * ss * hpre_ref[...],
      0.0,
  )
  h_ref[...] = h
  mxin = jnp.where(ss > 0.0, h, -jnp.inf)
  amx, asm = _rpartials(mxin, batch_ref[...])
  pmx_ref[...] = amx.reshape(1, G, D)
  psm_ref[...] = asm.reshape(1, G, D)


def _k_combine3(sa, sb, hpre, dinv, selfsel, batchc):
  return pl.pallas_call(
      _combine3_body,
      grid=(NRB,),
      in_specs=[
          pl.BlockSpec((RB, D), lambda i: (i, 0)),
          pl.BlockSpec((RB, D), lambda i: (i, 0)),
          pl.BlockSpec((RB, D), lambda i: (i, 0)),
          pl.BlockSpec((RB, 1), lambda i: (i, 0)),
          pl.BlockSpec((RB, 1), lambda i: (i, 0)),
          pl.BlockSpec((RB, 1), lambda i: (i, 0)),
      ],
      out_specs=[
          pl.BlockSpec((RB, D), lambda i: (i, 0)),
          pl.BlockSpec((1, G, D), lambda i: (i, 0, 0)),
          pl.BlockSpec((1, G, D), lambda i: (i, 0, 0)),
      ],
      out_shape=[
          jax.ShapeDtypeStruct((NPAD, D), f32),
          jax.ShapeDtypeStruct((NRB, G, D), f32),
          jax.ShapeDtypeStruct((NRB, G, D), f32),
      ],
  )(sa, sb, hpre, dinv, selfsel, batchc)


def _mmgate_body(h_ref, sel_ref, gate_ref, w_ref, b_ref, batch_ref, hg_ref,
                 hn_ref, pmx_ref, psm_ref):
  sel = sel_ref[...]
  hg = sel * gate_ref[...] * h_ref[...]
  hg_ref[...] = hg
  hn_ref[...] = (
      jnp.dot(hg, w_ref[...], preferred_element_type=f32) + b_ref[...]
  )
  mxin = jnp.where(sel > 0.0, hg, -jnp.inf)
  amx, asm = _rpartials(mxin, batch_ref[...])
  pmx_ref[...] = amx.reshape(1, G, D)
  psm_ref[...] = asm.reshape(1, G, D)


def _k_mmgate(h, sel, gate, w, b, batchc):
  return pl.pallas_call(
      _mmgate_body,
      grid=(NRB,),
      in_specs=[
          pl.BlockSpec((RB, D), lambda i: (i, 0)),
          pl.BlockSpec((RB, 1), lambda i: (i, 0)),
          pl.BlockSpec((RB, 1), lambda i: (i, 0)),
          pl.BlockSpec((D, D), lambda i: (0, 0)),
          pl.BlockSpec((1, D), lambda i: (0, 0)),
          pl.BlockSpec((RB, 1), lambda i: (i, 0)),
      ],
      out_specs=[
          pl.BlockSpec((RB, D), lambda i: (i, 0)),
          pl.BlockSpec((RB, D), lambda i: (i, 0)),
          pl.BlockSpec((1, G, D), lambda i: (i, 0, 0)),
          pl.BlockSpec((1, G, D), lambda i: (i, 0, 0)),
      ],
      out_shape=[
          jax.ShapeDtypeStruct((NPAD, D), f32),
          jax.ShapeDtypeStruct((NPAD, D), f32),
          jax.ShapeDtypeStruct((NRB, G, D), f32),
          jax.ShapeDtypeStruct((NRB, G, D), f32),
      ],
  )(h, sel, gate, w, b, batchc)


def _score_body(k, spa_ref, spb_ref, ppr_ref, dinv_ref, ss_ref, prev_ref,
                batch_ref, sel_ref, gate_ref, cnt_ref):
  dinv = dinv_ref[...]
  score = jnp.tanh(
      dinv * (spa_ref[...] + spb_ref[...]) + dinv * ss_ref[...] * ppr_ref[...]
  )
  gate_ref[...] = score
  cand = prev_ref[...] > 0.0

  b = lax.bitcast_convert_type(score, i32)
  m = b >> 31
  u = b ^ (m | jnp.int32(-(2**31)))
  u = jnp.where(cand, u, jnp.int32(0))

  prefix = jnp.int32(0)
  kk = jnp.int32(k)
  for j in range(31, -1, -1):
    bit = _to_i32(1 << j)
    hm = _to_i32(~((1 << j) - 1))
    meq = (u & hm) == (prefix | bit)
    cnt1 = jnp.sum(meq.astype(i32))
    take = cnt1 >= kk
    prefix = jnp.where(take, prefix | bit, prefix)
    kk = jnp.where(take, kk, kk - cnt1)

  bias = jnp.int32(-(2**31))
  ugt = (u ^ bias) > (prefix ^ bias)
  tie = cand & (u == prefix)
  need = jnp.int32(k) - jnp.sum(ugt.astype(i32))

  idx = (
      lax.broadcasted_iota(i32, (NPAD // 128, 128), 0) * 128
      + lax.broadcasted_iota(i32, (NPAD // 128, 128), 1)
  )
  lo = jnp.int32(0)
  hi = jnp.int32(NPAD)
  for _ in range(14):
    mid = (lo + hi) // 2
    cm = jnp.sum((tie & (idx < mid)).astype(i32))
    ok = cm >= need
    hi = jnp.where(ok, mid, hi)
    lo = jnp.where(ok, lo, mid + 1)

  sel = (ugt | (tie & (idx < lo))).astype(f32)
  sel_ref[...] = sel

  batch = batch_ref[...]
  lane = lax.broadcasted_iota(i32, (1, 128), 1)
  acc = jnp.zeros((1, 128), f32)
  for g in range(G):
    cg = jnp.sum(jnp.where(batch == g, sel, 0.0))
    acc = acc + jnp.where(lane == g, cg, 0.0)
  cnt_ref[...] = acc


def _k_score(k, spa, spb, ppr, dinv, selfsel, prevsel, batch2):
  r = NPAD // 128
  return pl.pallas_call(
      functools.partial(_score_body, k),
      out_shape=[
          jax.ShapeDtypeStruct((r, 128), f32),
          jax.ShapeDtypeStruct((r, 128), f32),
          jax.ShapeDtypeStruct((1, 128), f32),
      ],
  )(spa, spb, ppr, dinv, selfsel, prevsel, batch2)


def _final_body(pmx1_ref, psm1_ref, pmx2_ref, psm2_ref, pmx3_ref, psm3_ref,
                cnt1_ref, cnt2_ref, w1_ref, b1_ref, w2_ref, b2_ref, w3_ref,
                b3_ref, o_ref):
  def readout(pmx_ref, psm_ref, cnt_ref):
    mx = jnp.max(pmx_ref[...], axis=0)
    sm = jnp.sum(psm_ref[...], axis=0)
    mx = jnp.where(mx > -jnp.inf, mx, 0.0)
    cnt = cnt_ref[...].reshape(1, 128)[:, :G].reshape(G, 1)
    mean = sm / jnp.maximum(cnt, 1.0)
    return jnp.concatenate([mx, mean], axis=1)

  x1 = readout(pmx1_ref, psm1_ref, cnt1_ref)
  x2 = readout(pmx2_ref, psm2_ref, cnt2_ref)
  x3 = readout(pmx3_ref, psm3_ref, cnt2_ref)
  z = (
      jnp.maximum(x1, 0.0) + jnp.maximum(x2, 0.0) + jnp.maximum(x3, 0.0)
  )
  z = jnp.maximum(
      jnp.dot(z, w1_ref[...], preferred_element_type=f32) + b1_ref[...], 0.0
  )
  z = jnp.maximum(
      jnp.dot(z, w2_ref[...], preferred_element_type=f32) + b2_ref[...], 0.0
  )
  y = jnp.dot(z, w3_ref[...], preferred_element_type=f32) + b3_ref[...]
  m = jnp.max(y, axis=-1, keepdims=True)
  lse = jnp.log(jnp.sum(jnp.exp(y - m), axis=-1, keepdims=True)) + m
  o_ref[...] = y - lse


def _k_final(r1, r2, r3, cnt1, cnt2, w1, b1, w2, b2, w3, b3):
  ncls = w3.shape[1]
  return pl.pallas_call(
      _final_body,
      out_shape=jax.ShapeDtypeStruct((G, ncls), f32),
  )(r1[0], r1[1], r2[0], r2[1], r3[0], r3[1], cnt1, cnt2, w1, b1, w2, b2,
    w3, b3)


def _sc_mesh():
  return plsc.VectorSubcoreMesh(core_axis_name="c", subcore_axis_name="s")


def _sc_deg(src, dst, sel):
  mesh = _sc_mesh()
  nc, ns = mesh.num_cores, mesh.num_subcores
  nw = nc * ns
  epw = E // nw
  ppt = ((epw + 127) // 128) * 128 + 256

  @functools.partial(
      pl.kernel,
      out_type=(
          jax.ShapeDtypeStruct((nc, NPAD // 128, 128), f32),
          jax.ShapeDtypeStruct((nw, ppt), i32),
          jax.ShapeDtypeStruct((nw, 16), i32),
      ),
      mesh=mesh,
      scratch_types=[
          pltpu.VMEM((NPAD // 128, 128), f32),
          pltpu.VMEM((NPAD // 128, 128), f32),
          pltpu.VMEM((epw,), i32),
          pltpu.VMEM((epw,), i32),
          pltpu.VMEM((ppt,), i32),
          pltpu.VMEM((16,), i32),
          pltpu.VMEM((NPAD // 128,), i32),
          pltpu.VMEM_SHARED((NPAD // 128, 128), f32),
      ],
      compiler_params=pltpu.CompilerParams(needs_layout_passes=False),
  )
  def kfn(src_hbm, dst_hbm, sel_hbm, deg_hbm, pk_hbm, cnt_hbm, sel_v, acc_v,
          src_v, dst_v, pk_v, cnt_v, rix_v, sacc):
    c = lax.axis_index("c")
    s = lax.axis_index("s")
    wid = c * ns + s
    base = wid * epw
    pltpu.sync_copy(sel_hbm, sel_v)
    pltpu.sync_copy(src_hbm.at[pl.ds(base, epw)], src_v)
    pltpu.sync_copy(dst_hbm.at[pl.ds(base, epw)], dst_v)

    def zb(i, _):
      acc_v[i // 8, pl.ds((i % 8) * 16, 16)] = jnp.zeros((16,), f32)
      return 0

    lax.fori_loop(0, NPAD // 16, zb, 0)

    def zi(i, _):
      rix_v[pl.ds(i * 16, 16)] = lax.iota(i32, 16) + i * 16
      return 0

    lax.fori_loop(0, NPAD // 128 // 16, zi, 0)

    @pl.when(s == 0)
    def _():
      pltpu.sync_copy(acc_v, sacc)

    plsc.subcore_barrier()

    def body(i, cursor):
      sv = src_v[pl.ds(i * 16, 16)]
      dv = dst_v[pl.ds(i * 16, 16)]
      valid = plsc.load_gather(sel_v, [sv >> 7, sv & 127]) * plsc.load_gather(
          sel_v, [dv >> 7, dv & 127]
      )
      ok = valid > 0.0
      mdst = jnp.where(ok, dv, jnp.int32(TRASH))
      plsc.store_compressed(pk_v.at[pl.ds(cursor, 16)], sv | (mdst << 16), mask=ok)
      plsc.addupdate_scatter(acc_v, [mdst >> 7, mdst & 127], valid)
      return cursor + jnp.max(plsc.all_reduce_population_count(ok))

    cnt = lax.fori_loop(0, epw // 16, body, jnp.int32(0))

    def pad(l, _):
      pk_v[pl.ds(cnt + l * 16, 16)] = jnp.full((16,), TRASH << 16, i32)
      return 0

    lax.fori_loop(0, 16, pad, 0)
    cnt_v[...] = jnp.full((16,), 1, i32) * cnt
    pltpu.sync_copy(cnt_v, cnt_hbm.at[wid])
    pltpu.sync_copy(pk_v, pk_hbm.at[wid])
    pltpu.sync_copy(acc_v, sacc.at[rix_v], add=True)
    plsc.subcore_barrier()

    @pl.when(s == 0)
    def _():
      pltpu.sync_copy(sacc, deg_hbm.at[c])

  deg, pk, cnts = kfn(src, dst, sel)
  return deg, pk.reshape(nw, ppt // 128, 128), cnts


def _sc_p(ppr, pk, cnts):
  mesh = _sc_mesh()
  nc, ns = mesh.num_cores, mesh.num_subcores
  nw = nc * ns
  ppt = pk.shape[1] * 128

  @functools.partial(
      pl.kernel,
      out_type=jax.ShapeDtypeStruct((nc, NPAD // 128, 128), f32),
      mesh=mesh,
      scratch_types=[
          pltpu.VMEM((NPAD // 128, 128), f32),
          pltpu.VMEM((NPAD // 128, 128), f32),
          pltpu.VMEM((ppt // 128, 128), i32),
          pltpu.VMEM((16,), i32),
          pltpu.VMEM((NPAD // 128,), i32),
          pltpu.VMEM_SHARED((NPAD // 128, 128), f32),
      ],
      compiler_params=pltpu.CompilerParams(needs_layout_passes=False),
  )
  def kfn(ppr_hbm, pk_hbm, cnt_hbm, out_hbm, ppr_v, acc_v, pk_v, cnt_v,
          rix_v, sacc):
    c = lax.axis_index("c")
    s = lax.axis_index("s")
    wid = c * ns + s
    pltpu.sync_copy(ppr_hbm, ppr_v)
    pltpu.sync_copy(pk_hbm.at[wid], pk_v)
    pltpu.sync_copy(cnt_hbm.at[wid], cnt_v)

    def zb(i, _):
      acc_v[i // 8, pl.ds((i % 8) * 16, 16)] = jnp.zeros((16,), f32)
      return 0

    lax.fori_loop(0, NPAD // 16, zb, 0)

    def zi(i, _):
      rix_v[pl.ds(i * 16, 16)] = lax.iota(i32, 16) + i * 16
      return 0

    lax.fori_loop(0, NPAD // 128 // 16, zi, 0)

    @pl.when(s == 0)
    def _():
      pltpu.sync_copy(acc_v, sacc)

    plsc.subcore_barrier()
    cnt = jnp.max(cnt_v[pl.ds(0, 16)])
    n16 = (cnt + 15) // 16

    def body(i, _):
      pkv = pk_v[i >> 3, pl.ds((i & 7) * 16, 16)]
      sv = pkv & 16383
      mv = pkv >> 16
      val = plsc.load_gather(ppr_v, [sv >> 7, sv & 127])
      plsc.addupdate_scatter(acc_v, [mv >> 7, mv & 127], val)
      return 0

    lax.fori_loop(0, n16, body, 0)
    pltpu.sync_copy(acc_v, sacc.at[rix_v], add=True)
    plsc.subcore_barrier()

    @pl.when(s == 0)
    def _():
      pltpu.sync_copy(sacc, out_hbm.at[c])

  return kfn(ppr, pk, cnts)


def _sc_feat(hprime, pk, cnts):
  mesh = _sc_mesh()
  nc, ns = mesh.num_cores, mesh.num_subcores
  nw = nc * ns
  fch = 128
  ppt = pk.shape[1] * 128
  rps = NPAD // ns

  @functools.partial(
      pl.kernel,
      out_type=jax.ShapeDtypeStruct((nc, NPAD, D), f32),
      mesh=mesh,
      scratch_types=[
          pltpu.VMEM((ppt // 128, 128), i32),
          pltpu.VMEM((16,), i32),
          pltpu.VMEM((fch,), i32),
          pltpu.VMEM((fch,), i32),
          pltpu.VMEM((fch,), i32),
          pltpu.VMEM((fch,), i32),
          pltpu.VMEM((fch, D), f32),
          pltpu.VMEM((fch, D), f32),
          pltpu.VMEM_SHARED((NPAD, D), f32),
          pltpu.SemaphoreType.DMA,
          pltpu.SemaphoreType.DMA,
      ],
      compiler_params=pltpu.CompilerParams(needs_layout_passes=False),
  )
  def kfn(h_hbm, pk_hbm, cnt_hbm, out_hbm, idxp_v, cnt_v, is0, id0, is1, id1,
          rows0, rows1, sacc, sem0, sem1):
    c = lax.axis_index("c")
    s = lax.axis_index("s")
    wid = c * ns + s

    def zr(i, _):
      def zc(l, _):
        rows0[i, pl.ds(l * 16, 16)] = jnp.zeros((16,), f32)
        return 0

      lax.fori_loop(0, D // 16, zc, 0)
      return 0

    lax.fori_loop(0, fch, zr, 0)

    def zs(j, _):
      pltpu.sync_copy(rows0, sacc.at[pl.ds(s * rps + j * fch, fch), :])
      return 0

    lax.fori_loop(0, rps // fch, zs, 0)
    pltpu.sync_copy(pk_hbm.at[wid], idxp_v)
    pltpu.sync_copy(cnt_hbm.at[wid], cnt_v)
    plsc.subcore_barrier()
    cnt = jnp.max(cnt_v[pl.ds(0, 16)])
    n2 = jnp.maximum((cnt + 2 * fch - 1) // (2 * fch), 1)

    def unpack(ci, isb, idb):
      def ul(l, _):
        pos = ci * fch + l * 16
        v = idxp_v[pos >> 7, pl.ds(pos & 127, 16)]
        isb[pl.ds(l * 16, 16)] = v & 16383
        idb[pl.ds(l * 16, 16)] = v >> 16
        return 0

      lax.fori_loop(0, fch // 16, ul, 0)

    unpack(0, is0, id0)
    pltpu.async_copy(h_hbm.at[is0], rows0, sem0)

    def it2(cj, _):
      ci0 = cj * 2
      unpack(ci0 + 1, is1, id1)
      pltpu.async_copy(h_hbm.at[is1], rows1, sem1)
      pltpu.make_async_copy(h_hbm.at[is0], rows0, sem0).wait()
      pltpu.sync_copy(rows0, sacc.at[id0], add=True)

      @pl.when(ci0 + 2 < n2 * 2)
      def _():
        unpack(ci0 + 2, is0, id0)
        pltpu.async_copy(h_hbm.at[is0], rows0, sem0)

      pltpu.make_async_copy(h_hbm.at[is1], rows1, sem1).wait()
      pltpu.sync_copy(rows1, sacc.at[id1], add=True)
      return 0

    lax.fori_loop(0, n2, it2, 0)
    plsc.subcore_barrier()
    pltpu.sync_copy(
        sacc.at[pl.ds(s * rps, rps), :], out_hbm.at[c, pl.ds(s * rps, rps), :]
    )

  return kfn(hprime, pk, cnts)


def kernel(x, edge_index, batch, neg_num, samp_bias1, samp_bias2, W1, b1, W2,
           b2, W3, b3, Wp1, bp1, Wp2, bp2, Wl1, bl1, Wl2, bl2, Wl3, bl3):
  del neg_num, samp_bias1, samp_bias2
  src = edge_index[0]
  dst = edge_index[1]

  xp = jnp.pad(x, ((0, NPAD - N), (0, 0)))
  batchp = jnp.pad(batch, (0, NPAD - N), constant_values=127)
  batch2 = batchp.reshape(-1, 128)
  batchc = batchp.reshape(NPAD, 1)
  rowvalid = (jnp.arange(NPAD) < N).astype(f32)
  rv1 = rowvalid.reshape(NPAD, 1)
  rv2 = rowvalid.reshape(-1, 128)

  k1 = int(np.ceil(0.5 * N))
  k2 = int(np.ceil(0.5 * k1))

  h1pre = _k_mm(xp, W1, b1.reshape(1, -1))
  deg0, pk0, cnt0 = _sc_deg(src, dst, rv2)
  dinv0, h1prime = _k_fold(
      h1pre, deg0[0].reshape(NPAD, 1), deg0[1].reshape(NPAD, 1), rv1
  )
  S1 = _sc_feat(h1prime, pk0, cnt0)
  h1, p1prime = _k_combine(S1[0], S1[1], h1pre, dinv0, rv1, Wp1, bp1.reshape(1, 1))
  Sp1 = _sc_p(p1prime.reshape(-1, 128), pk0, cnt0)
  sel1f, gate1f, cnts1 = _k_score(
      k1,
      Sp1[0].reshape(-1, 128),
      Sp1[1].reshape(-1, 128),
      p1prime.reshape(-1, 128),
      dinv0.reshape(-1, 128),
      rv2,
      rv2,
      batch2,
  )
  sel1 = sel1f.reshape(NPAD, 1)
  gate1 = gate1f.reshape(NPAD, 1)
  h1p, h2pre, pmx1, psm1 = _k_mmgate(
      h1, sel1, gate1, W2, b2.reshape(1, -1), batchc
  )

  deg1, pk1, cnt1 = _sc_deg(src, dst, sel1f)
  dinv1, h2prime = _k_fold(
      h2pre, deg1[0].reshape(NPAD, 1), deg1[1].reshape(NPAD, 1), sel1
  )
  S2 = _sc_feat(h2prime, pk1, cnt1)
  h2, p2prime = _k_combine(S2[0], S2[1], h2pre, dinv1, sel1, Wp2, bp2.reshape(1, 1))
  Sp2 = _sc_p(p2prime.reshape(-1, 128), pk1, cnt1)
  sel2f, gate2f, cnts2 = _k_score(
      k2,
      Sp2[0].reshape(-1, 128),
      Sp2[1].reshape(-1, 128),
      p2prime.reshape(-1, 128),
      dinv1.reshape(-1, 128),
      sel1f,
      sel1f,
      batch2,
  )
  sel2 = sel2f.reshape(NPAD, 1)
  gate2 = gate2f.reshape(NPAD, 1)
  h2p, h3pre, pmx2, psm2 = _k_mmgate(
      h2, sel2, gate2, W3, b3.reshape(1, -1), batchc
  )

  deg2, pk2, cnt2 = _sc_deg(src, dst, sel2f)
  dinv2, h3prime = _k_fold(
      h3pre, deg2[0].reshape(NPAD, 1), deg2[1].reshape(NPAD, 1), sel2
  )
  S3 = _sc_feat(h3prime, pk2, cnt2)
  h3, pmx3, psm3 = _k_combine3(S3[0], S3[1], h3pre, dinv2, sel2, batchc)

  return _k_final(
      (pmx1, psm1),
      (pmx2, psm2),
      (pmx3, psm3),
      cnts1,
      cnts2,
      Wl1,
      bl1.reshape(1, -1),
      Wl2,
      bl2.reshape(1, -1),
      Wl3,
      bl3.reshape(1, -1),
  )


# speedup vs baseline: 1.4577x; 1.4577x over previous
import functools

import jax
import jax.numpy as jnp
import numpy as np
from jax import lax
from jax.experimental import pallas as pl
import jax.experimental.pallas.tpu as pltpu
from jax.experimental.pallas import tpu_sc as plsc

N = 10000
NPAD = 10240
E = 320000
D = 128
G = 64
TRASH = N
RB = 512
NRB = NPAD // RB
CR = 256

f32 = jnp.float32
i32 = jnp.int32


def _to_i32(v):
  v &= 0xFFFFFFFF
  return jnp.int32(v - 2**32 if v >= 2**31 else v)


def _mm_body(x_ref, w_ref, b_ref, o_ref):
  o_ref[...] = (
      jnp.dot(x_ref[...], w_ref[...], preferred_element_type=f32) + b_ref[...]
  )


def _k_mm(x, w, b):
  dout = w.shape[1]
  return pl.pallas_call(
      _mm_body,
      grid=(NRB,),
      in_specs=[
          pl.BlockSpec((RB, D), lambda i: (i, 0)),
          pl.BlockSpec((D, dout), lambda i: (0, 0)),
          pl.BlockSpec((1, dout), lambda i: (0, 0)),
      ],
      out_specs=pl.BlockSpec((RB, dout), lambda i: (i, 0)),
      out_shape=jax.ShapeDtypeStruct((NPAD, dout), f32),
  )(x, w, b)


def _fold_body(hpre_ref, da_ref, db_ref, ss_ref, dinv_ref, hp_ref):
  deg = da_ref[...] + db_ref[...] + ss_ref[...]
  dinv = lax.rsqrt(jnp.clip(deg, 1e-12, None))
  dinv_ref[...] = dinv
  hp_ref[...] = dinv * hpre_ref[...]


def _k_fold(hpre, dega, degb, selfsel):
  return pl.pallas_call(
      _fold_body,
      grid=(NRB,),
      in_specs=[
          pl.BlockSpec((RB, D), lambda i: (i, 0)),
          pl.BlockSpec((RB, 1), lambda i: (i, 0)),
          pl.BlockSpec((RB, 1), lambda i: (i, 0)),
          pl.BlockSpec((RB, 1), lambda i: (i, 0)),
      ],
      out_specs=[
          pl.BlockSpec((RB, 1), lambda i: (i, 0)),
          pl.BlockSpec((RB, D), lambda i: (i, 0)),
      ],
      out_shape=[
          jax.ShapeDtypeStruct((NPAD, 1), f32),
          jax.ShapeDtypeStruct((NPAD, D), f32),
      ],
  )(hpre, dega, degb, selfsel)


def _combine_body(sa_ref, sb_ref, hpre_ref, dinv_ref, ss_ref, wp_ref, bp_ref,
                  h_ref, pp_ref):
  dinv = dinv_ref[...]
  ss = ss_ref[...]
  h = jnp.maximum(
      dinv * (sa_ref[...] + sb_ref[...]) + dinv * dinv * ss * hpre_ref[...],
      0.0,
  )
  h_ref[...] = h
  pp_ref[...] = dinv * (
      jnp.dot(h, wp_ref[...], preferred_element_type=f32) + bp_ref[...]
  )


def _k_combine(sa, sb, hpre, dinv, selfsel, wp, bp):
  return pl.pallas_call(
      _combine_body,
      grid=(NRB,),
      in_specs=[
          pl.BlockSpec((RB, D), lambda i: (i, 0)),
          pl.BlockSpec((RB, D), lambda i: (i, 0)),
          pl.BlockSpec((RB, D), lambda i: (i, 0)),
          pl.BlockSpec((RB, 1), lambda i: (i, 0)),
          pl.BlockSpec((RB, 1), lambda i: (i, 0)),
          pl.BlockSpec((D, 1), lambda i: (0, 0)),
          pl.BlockSpec((1, 1), lambda i: (0, 0)),
      ],
      out_specs=[
          pl.BlockSpec((RB, D), lambda i: (i, 0)),
          pl.BlockSpec((RB, 1), lambda i: (i, 0)),
      ],
      out_shape=[
          jax.ShapeDtypeStruct((NPAD, D), f32),
          jax.ShapeDtypeStruct((NPAD, 1), f32),
      ],
  )(sa, sb, hpre, dinv, selfsel, wp, bp)


def _rpartials(mxin, bcol):
  g0 = bcol[0, 0]
  g1 = jnp.minimum(bcol[RB - 1, 0], G - 1)
  sumv = jnp.where(mxin > -jnp.inf, mxin, 0.0)
  giota = lax.broadcasted_iota(i32, (G, 1), 0)

  def body(g, carry):
    amx, asm = carry
    m = bcol == g
    mx = jnp.max(jnp.where(m, mxin, -jnp.inf), axis=0, keepdims=True)
    sm = jnp.sum(jnp.where(m, sumv, 0.0), axis=0, keepdims=True)
    amx = jnp.where(giota == g, mx, amx)
    asm = jnp.where(giota == g, sm, asm)
    return amx, asm

  amx0 = jnp.full((G, D), -jnp.inf, f32)
  asm0 = jnp.zeros((G, D), f32)
  return lax.fori_loop(g0, g1 + 1, body, (amx0, asm0))


def _combine3_body(sa_ref, sb_ref, hpre_ref, dinv_ref, ss_ref, batch_ref,
                   h_ref, pmx_ref, psm_ref):
  dinv = dinv_ref[...]
  ss = ss_ref[...]
  h = jnp.maximum(
      dinv * (sa_ref[...] + sb_ref[...]) + dinv * dinv * ss * hpre_ref[...],
      0.0,
  )
  h_ref[...] = h
  mxin = jnp.where(ss > 0.0, h, -jnp.inf)
  amx, asm = _rpartials(mxin, batch_ref[...])
  pmx_ref[...] = amx.reshape(1, G, D)
  psm_ref[...] = asm.reshape(1, G, D)


def _k_combine3(sa, sb, hpre, dinv, selfsel, batchc):
  return pl.pallas_call(
      _combine3_body,
      grid=(NRB,),
      in_specs=[
          pl.BlockSpec((RB, D), lambda i: (i, 0)),
          pl.BlockSpec((RB, D), lambda i: (i, 0)),
          pl.BlockSpec((RB, D), lambda i: (i, 0)),
          pl.BlockSpec((RB, 1), lambda i: (i, 0)),
          pl.BlockSpec((RB, 1), lambda i: (i, 0)),
          pl.BlockSpec((RB, 1), lambda i: (i, 0)),
      ],
      out_specs=[
          pl.BlockSpec((RB, D), lambda i: (i, 0)),
          pl.BlockSpec((1, G, D), lambda i: (i, 0, 0)),
          pl.BlockSpec((1, G, D), lambda i: (i, 0, 0)),
      ],
      out_shape=[
          jax.ShapeDtypeStruct((NPAD, D), f32),
          jax.ShapeDtypeStruct((NRB, G, D), f32),
          jax.ShapeDtypeStruct((NRB, G, D), f32),
      ],
  )(sa, sb, hpre, dinv, selfsel, batchc)


def _mmgate_body(h_ref, sel_ref, gate_ref, w_ref, b_ref, batch_ref, hg_ref,
                 hn_ref, pmx_ref, psm_ref):
  sel = sel_ref[...]
  hg = sel * gate_ref[...] * h_ref[...]
  hg_ref[...] = hg
  hn_ref[...] = (
      jnp.dot(hg, w_ref[...], preferred_element_type=f32) + b_ref[...]
  )
  mxin = jnp.where(sel > 0.0, hg, -jnp.inf)
  amx, asm = _rpartials(mxin, batch_ref[...])
  pmx_ref[...] = amx.reshape(1, G, D)
  psm_ref[...] = asm.reshape(1, G, D)


def _k_mmgate(h, sel, gate, w, b, batchc):
  return pl.pallas_call(
      _mmgate_body,
      grid=(NRB,),
      in_specs=[
          pl.BlockSpec((RB, D), lambda i: (i, 0)),
          pl.BlockSpec((RB, 1), lambda i: (i, 0)),
          pl.BlockSpec((RB, 1), lambda i: (i, 0)),
          pl.BlockSpec((D, D), lambda i: (0, 0)),
          pl.BlockSpec((1, D), lambda i: (0, 0)),
          pl.BlockSpec((RB, 1), lambda i: (i, 0)),
      ],
      out_specs=[
          pl.BlockSpec((RB, D), lambda i: (i, 0)),
          pl.BlockSpec((RB, D), lambda i: (i, 0)),
          pl.BlockSpec((1, G, D), lambda i: (i, 0, 0)),
          pl.BlockSpec((1, G, D), lambda i: (i, 0, 0)),
      ],
      out_shape=[
          jax.ShapeDtypeStruct((NPAD, D), f32),
          jax.ShapeDtypeStruct((NPAD, D), f32),
          jax.ShapeDtypeStruct((NRB, G, D), f32),
          jax.ShapeDtypeStruct((NRB, G, D), f32),
      ],
  )(h, sel, gate, w, b, batchc)


def _score_body(k, spa_ref, spb_ref, ppr_ref, dinv_ref, ss_ref, prev_ref,
                batch_ref, sel_ref, gate_ref, cnt_ref):
  dinv = dinv_ref[...]
  score = jnp.tanh(
      dinv * (spa_ref[...] + spb_ref[...]) + dinv * ss_ref[...] * ppr_ref[...]
  )
  gate_ref[...] = score
  cand = prev_ref[...] > 0.0

  b = lax.bitcast_convert_type(score, i32)
  m = b >> 31
  u = b ^ (m | jnp.int32(-(2**31)))
  u = jnp.where(cand, u, jnp.int32(0))

  prefix = jnp.int32(0)
  kk = jnp.int32(k)
  for j in range(31, -1, -1):
    bit = _to_i32(1 << j)
    hm = _to_i32(~((1 << j) - 1))
    meq = (u & hm) == (prefix | bit)
    cnt1 = jnp.sum(meq.astype(i32))
    take = cnt1 >= kk
    prefix = jnp.where(take, prefix | bit, prefix)
    kk = jnp.where(take, kk, kk - cnt1)

  bias = jnp.int32(-(2**31))
  ugt = (u ^ bias) > (prefix ^ bias)
  tie = cand & (u == prefix)
  need = jnp.int32(k) - jnp.sum(ugt.astype(i32))

  idx = (
      lax.broadcasted_iota(i32, (NPAD // 128, 128), 0) * 128
      + lax.broadcasted_iota(i32, (NPAD // 128, 128), 1)
  )
  lo = jnp.int32(0)
  hi = jnp.int32(NPAD)
  for _ in range(14):
    mid = (lo + hi) // 2
    cm = jnp.sum((tie & (idx < mid)).astype(i32))
    ok = cm >= need
    hi = jnp.where(ok, mid, hi)
    lo = jnp.where(ok, lo, mid + 1)

  sel = (ugt | (tie & (idx < lo))).astype(f32)
  sel_ref[...] = sel

  batch = batch_ref[...]
  lane = lax.broadcasted_iota(i32, (1, 128), 1)
  acc = jnp.zeros((1, 128), f32)
  for g in range(G):
    cg = jnp.sum(jnp.where(batch == g, sel, 0.0))
    acc = acc + jnp.where(lane == g, cg, 0.0)
  cnt_ref[...] = acc


def _k_score(k, spa, spb, ppr, dinv, selfsel, prevsel, batch2):
  r = NPAD // 128
  return pl.pallas_call(
      functools.partial(_score_body, k),
      out_shape=[
          jax.ShapeDtypeStruct((r, 128), f32),
          jax.ShapeDtypeStruct((r, 128), f32),
          jax.ShapeDtypeStruct((1, 128), f32),
      ],
  )(spa, spb, ppr, dinv, selfsel, prevsel, batch2)


def _final_body(pmx1_ref, psm1_ref, pmx2_ref, psm2_ref, pmx3_ref, psm3_ref,
                cnt1_ref, cnt2_ref, w1_ref, b1_ref, w2_ref, b2_ref, w3_ref,
                b3_ref, o_ref):
  def readout(pmx_ref, psm_ref, cnt_ref):
    mx = jnp.max(pmx_ref[...], axis=0)
    sm = jnp.sum(psm_ref[...], axis=0)
    mx = jnp.where(mx > -jnp.inf, mx, 0.0)
    cnt = cnt_ref[...].reshape(1, 128)[:, :G].reshape(G, 1)
    mean = sm / jnp.maximum(cnt, 1.0)
    return jnp.concatenate([mx, mean], axis=1)

  x1 = readout(pmx1_ref, psm1_ref, cnt1_ref)
  x2 = readout(pmx2_ref, psm2_ref, cnt2_ref)
  x3 = readout(pmx3_ref, psm3_ref, cnt2_ref)
  z = (
      jnp.maximum(x1, 0.0) + jnp.maximum(x2, 0.0) + jnp.maximum(x3, 0.0)
  )
  z = jnp.maximum(
      jnp.dot(z, w1_ref[...], preferred_element_type=f32) + b1_ref[...], 0.0
  )
  z = jnp.maximum(
      jnp.dot(z, w2_ref[...], preferred_element_type=f32) + b2_ref[...], 0.0
  )
  y = jnp.dot(z, w3_ref[...], preferred_element_type=f32) + b3_ref[...]
  m = jnp.max(y, axis=-1, keepdims=True)
  lse = jnp.log(jnp.sum(jnp.exp(y - m), axis=-1, keepdims=True)) + m
  o_ref[...] = y - lse


def _k_final(r1, r2, r3, cnt1, cnt2, w1, b1, w2, b2, w3, b3):
  ncls = w3.shape[1]
  return pl.pallas_call(
      _final_body,
      out_shape=jax.ShapeDtypeStruct((G, ncls), f32),
  )(r1[0], r1[1], r2[0], r2[1], r3[0], r3[1], cnt1, cnt2, w1, b1, w2, b2,
    w3, b3)


def _sc_mesh():
  return plsc.VectorSubcoreMesh(core_axis_name="c", subcore_axis_name="s")


def _sc_deg(src, dst, sel):
  mesh = _sc_mesh()
  nc, ns = mesh.num_cores, mesh.num_subcores
  nw = nc * ns
  epw = E // nw
  ppt = ((epw + 127) // 128) * 128 + 256

  @functools.partial(
      pl.kernel,
      out_type=(
          jax.ShapeDtypeStruct((nc, NPAD // 128, 128), f32),
          jax.ShapeDtypeStruct((nw, ppt), i32),
          jax.ShapeDtypeStruct((nw, 16), i32),
      ),
      mesh=mesh,
      scratch_types=[
          pltpu.VMEM((NPAD // 128, 128), f32),
          pltpu.VMEM((NPAD // 128, 128), f32),
          pltpu.VMEM((epw,), i32),
          pltpu.VMEM((epw,), i32),
          pltpu.VMEM((ppt,), i32),
          pltpu.VMEM((16,), i32),
          pltpu.VMEM((NPAD // 128,), i32),
          pltpu.VMEM_SHARED((NPAD // 128, 128), f32),
      ],
      compiler_params=pltpu.CompilerParams(needs_layout_passes=False),
  )
  def kfn(src_hbm, dst_hbm, sel_hbm, deg_hbm, pk_hbm, cnt_hbm, sel_v, acc_v,
          src_v, dst_v, pk_v, cnt_v, rix_v, sacc):
    c = lax.axis_index("c")
    s = lax.axis_index("s")
    wid = c * ns + s
    base = wid * epw
    pltpu.sync_copy(sel_hbm, sel_v)
    pltpu.sync_copy(src_hbm.at[pl.ds(base, epw)], src_v)
    pltpu.sync_copy(dst_hbm.at[pl.ds(base, epw)], dst_v)

    def zb(i, _):
      acc_v[i // 8, pl.ds((i % 8) * 16, 16)] = jnp.zeros((16,), f32)
      return 0

    lax.fori_loop(0, NPAD // 16, zb, 0)

    def zi(i, _):
      rix_v[pl.ds(i * 16, 16)] = lax.iota(i32, 16) + i * 16
      return 0

    lax.fori_loop(0, NPAD // 128 // 16, zi, 0)

    @pl.when(s == 0)
    def _():
      pltpu.sync_copy(acc_v, sacc)

    plsc.subcore_barrier()

    def body(i, cursor):
      sv = src_v[pl.ds(i * 16, 16)]
      dv = dst_v[pl.ds(i * 16, 16)]
      valid = plsc.load_gather(sel_v, [sv >> 7, sv & 127]) * plsc.load_gather(
          sel_v, [dv >> 7, dv & 127]
      )
      ok = valid > 0.0
      mdst = jnp.where(ok, dv, jnp.int32(TRASH))
      plsc.store_compressed(pk_v.at[pl.ds(cursor, 16)], sv | (mdst << 16), mask=ok)
      plsc.addupdate_scatter(acc_v, [mdst >> 7, mdst & 127], valid)
      return cursor + jnp.max(plsc.all_reduce_population_count(ok))

    cnt = lax.fori_loop(0, epw // 16, body, jnp.int32(0))

    def pad(l, _):
      pk_v[pl.ds(cnt + l * 16, 16)] = jnp.full((16,), TRASH << 16, i32)
      return 0

    lax.fori_loop(0, 16, pad, 0)
    cnt_v[...] = jnp.full((16,), 1, i32) * cnt
    pltpu.sync_copy(cnt_v, cnt_hbm.at[wid])
    pltpu.sync_copy(pk_v, pk_hbm.at[wid])
    pltpu.sync_copy(acc_v, sacc.at[rix_v], add=True)
    plsc.subcore_barrier()

    @pl.when(s == 0)
    def _():
      pltpu.sync_copy(sacc, deg_hbm.at[c])

  deg, pk, cnts = kfn(src, dst, sel)
  return deg, pk.reshape(nw, ppt // 128, 128), cnts


def _sc_p(ppr, pk, cnts):
  mesh = _sc_mesh()
  nc, ns = mesh.num_cores, mesh.num_subcores
  nw = nc * ns
  ppt = pk.shape[1] * 128

  @functools.partial(
      pl.kernel,
      out_type=jax.ShapeDtypeStruct((nc, NPAD // 128, 128), f32),
      mesh=mesh,
      scratch_types=[
          pltpu.VMEM((NPAD // 128, 128), f32),
          pltpu.VMEM((NPAD // 128, 128), f32),
          pltpu.VMEM((ppt // 128, 128), i32),
          pltpu.VMEM((16,), i32),
          pltpu.VMEM((NPAD // 128,), i32),
          pltpu.VMEM_SHARED((NPAD // 128, 128), f32),
      ],
      compiler_params=pltpu.CompilerParams(needs_layout_passes=False),
  )
  def kfn(ppr_hbm, pk_hbm, cnt_hbm, out_hbm, ppr_v, acc_v, pk_v, cnt_v,
          rix_v, sacc):
    c = lax.axis_index("c")
    s = lax.axis_index("s")
    wid = c * ns + s
    pltpu.sync_copy(ppr_hbm, ppr_v)
    pltpu.sync_copy(pk_hbm.at[wid], pk_v)
    pltpu.sync_copy(cnt_hbm.at[wid], cnt_v)

    def zb(i, _):
      acc_v[i // 8, pl.ds((i % 8) * 16, 16)] = jnp.zeros((16,), f32)
      return 0

    lax.fori_loop(0, NPAD // 16, zb, 0)

    def zi(i, _):
      rix_v[pl.ds(i * 16, 16)] = lax.iota(i32, 16) + i * 16
      return 0

    lax.fori_loop(0, NPAD // 128 // 16, zi, 0)

    @pl.when(s == 0)
    def _():
      pltpu.sync_copy(acc_v, sacc)

    plsc.subcore_barrier()
    cnt = jnp.max(cnt_v[pl.ds(0, 16)])
    n16 = (cnt + 15) // 16

    def body(i, _):
      pkv = pk_v[i >> 3, pl.ds((i & 7) * 16, 16)]
      sv = pkv & 16383
      mv = pkv >> 16
      val = plsc.load_gather(ppr_v, [sv >> 7, sv & 127])
      plsc.addupdate_scatter(acc_v, [mv >> 7, mv & 127], val)
      return 0

    lax.fori_loop(0, n16, body, 0)
    pltpu.sync_copy(acc_v, sacc.at[rix_v], add=True)
    plsc.subcore_barrier()

    @pl.when(s == 0)
    def _():
      pltpu.sync_copy(sacc, out_hbm.at[c])

  return kfn(ppr, pk, cnts)


def _sc_feat(hprime, pk, cnts):
  mesh = _sc_mesh()
  nc, ns = mesh.num_cores, mesh.num_subcores
  nw = nc * ns
  fch = 64
  ppt = pk.shape[1] * 128
  rps = NPAD // ns

  @functools.partial(
      pl.kernel,
      out_type=jax.ShapeDtypeStruct((nc, NPAD, D), f32),
      mesh=mesh,
      scratch_types=[
          pltpu.VMEM((ppt // 128, 128), i32),
          pltpu.VMEM((16,), i32),
          pltpu.VMEM((fch,), i32),
          pltpu.VMEM((fch,), i32),
          pltpu.VMEM((fch,), i32),
          pltpu.VMEM((fch,), i32),
          pltpu.VMEM((fch, D), f32),
          pltpu.VMEM((fch, D), f32),
          pltpu.VMEM_SHARED((NPAD, D), f32),
          pltpu.SemaphoreType.DMA,
          pltpu.SemaphoreType.DMA,
      ],
      compiler_params=pltpu.CompilerParams(needs_layout_passes=False),
  )
  def kfn(h_hbm, pk_hbm, cnt_hbm, out_hbm, idxp_v, cnt_v, is0, id0, is1, id1,
          rows0, rows1, sacc, sem0, sem1):
    c = lax.axis_index("c")
    s = lax.axis_index("s")
    wid = c * ns + s

    def zr(i, _):
      def zc(l, _):
        rows0[i, pl.ds(l * 16, 16)] = jnp.zeros((16,), f32)
        return 0

      lax.fori_loop(0, D // 16, zc, 0)
      return 0

    lax.fori_loop(0, fch, zr, 0)

    def zs(j, _):
      pltpu.sync_copy(rows0, sacc.at[pl.ds(s * rps + j * fch, fch), :])
      return 0

    lax.fori_loop(0, rps // fch, zs, 0)
    pltpu.sync_copy(pk_hbm.at[wid], idxp_v)
    pltpu.sync_copy(cnt_hbm.at[wid], cnt_v)
    plsc.subcore_barrier()
    cnt = jnp.max(cnt_v[pl.ds(0, 16)])
    n2 = jnp.maximum((cnt + 2 * fch - 1) // (2 * fch), 1)

    def unpack(ci, isb, idb):
      def ul(l, _):
        pos = ci * fch + l * 16
        v = idxp_v[pos >> 7, pl.ds(pos & 127, 16)]
        isb[pl.ds(l * 16, 16)] = v & 16383
        idb[pl.ds(l * 16, 16)] = v >> 16
        return 0

      lax.fori_loop(0, fch // 16, ul, 0)

    unpack(0, is0, id0)
    pltpu.async_copy(h_hbm.at[is0], rows0, sem0)

    def it2(cj, _):
      ci0 = cj * 2
      unpack(ci0 + 1, is1, id1)
      pltpu.async_copy(h_hbm.at[is1], rows1, sem1)
      pltpu.make_async_copy(h_hbm.at[is0], rows0, sem0).wait()
      pltpu.sync_copy(rows0, sacc.at[id0], add=True)

      @pl.when(ci0 + 2 < n2 * 2)
      def _():
        unpack(ci0 + 2, is0, id0)
        pltpu.async_copy(h_hbm.at[is0], rows0, sem0)

      pltpu.make_async_copy(h_hbm.at[is1], rows1, sem1).wait()
      pltpu.sync_copy(rows1, sacc.at[id1], add=True)
      return 0

    lax.fori_loop(0, n2, it2, 0)
    plsc.subcore_barrier()
    pltpu.sync_copy(
        sacc.at[pl.ds(s * rps, rps), :], out_hbm.at[c, pl.ds(s * rps, rps), :]
    )

  return kfn(hprime, pk, cnts)


def kernel(x, edge_index, batch, neg_num, samp_bias1, samp_bias2, W1, b1, W2,
           b2, W3, b3, Wp1, bp1, Wp2, bp2, Wl1, bl1, Wl2, bl2, Wl3, bl3):
  del neg_num, samp_bias1, samp_bias2
  src = edge_index[0]
  dst = edge_index[1]

  xp = jnp.pad(x, ((0, NPAD - N), (0, 0)))
  batchp = jnp.pad(batch, (0, NPAD - N), constant_values=127)
  batch2 = batchp.reshape(-1, 128)
  batchc = batchp.reshape(NPAD, 1)
  rowvalid = (jnp.arange(NPAD) < N).astype(f32)
  rv1 = rowvalid.reshape(NPAD, 1)
  rv2 = rowvalid.reshape(-1, 128)

  k1 = int(np.ceil(0.5 * N))
  k2 = int(np.ceil(0.5 * k1))

  h1pre = _k_mm(xp, W1, b1.reshape(1, -1))
  deg0, pk0, cnt0 = _sc_deg(src, dst, rv2)
  dinv0, h1prime = _k_fold(
      h1pre, deg0[0].reshape(NPAD, 1), deg0[1].reshape(NPAD, 1), rv1
  )
  S1 = _sc_feat(h1prime, pk0, cnt0)
  h1, p1prime = _k_combine(S1[0], S1[1], h1pre, dinv0, rv1, Wp1, bp1.reshape(1, 1))
  Sp1 = _sc_p(p1prime.reshape(-1, 128), pk0, cnt0)
  sel1f, gate1f, cnts1 = _k_score(
      k1,
      Sp1[0].reshape(-1, 128),
      Sp1[1].reshape(-1, 128),
      p1prime.reshape(-1, 128),
      dinv0.reshape(-1, 128),
      rv2,
      rv2,
      batch2,
  )
  sel1 = sel1f.reshape(NPAD, 1)
  gate1 = gate1f.reshape(NPAD, 1)
  h1p, h2pre, pmx1, psm1 = _k_mmgate(
      h1, sel1, gate1, W2, b2.reshape(1, -1), batchc
  )

  deg1, pk1, cnt1 = _sc_deg(src, dst, sel1f)
  dinv1, h2prime = _k_fold(
      h2pre, deg1[0].reshape(NPAD, 1), deg1[1].reshape(NPAD, 1), sel1
  )
  S2 = _sc_feat(h2prime, pk1, cnt1)
  h2, p2prime = _k_combine(S2[0], S2[1], h2pre, dinv1, sel1, Wp2, bp2.reshape(1, 1))
  Sp2 = _sc_p(p2prime.reshape(-1, 128), pk1, cnt1)
  sel2f, gate2f, cnts2 = _k_score(
      k2,
      Sp2[0].reshape(-1, 128),
      Sp2[1].reshape(-1, 128),
      p2prime.reshape(-1, 128),
      dinv1.reshape(-1, 128),
      sel1f,
      sel1f,
      batch2,
  )
  sel2 = sel2f.reshape(NPAD, 1)
  gate2 = gate2f.reshape(NPAD, 1)
  h2p, h3pre, pmx2, psm2 = _k_mmgate(
      h2, sel2, gate2, W3, b3.reshape(1, -1), batchc
  )

  deg2, pk2, cnt2 = _sc_deg(src, dst, sel2f)
  dinv2, h3prime = _k_fold(
      h3pre, deg2[0].reshape(NPAD, 1), deg2[1].reshape(NPAD, 1), sel2
  )
  S3 = _sc_feat(h3prime, pk2, cnt2)
  h3, pmx3, psm3 = _k_combine3(S3[0], S3[1], h3pre, dinv2, sel2, batchc)

  return _k_final(
      (pmx1, psm1),
      (pmx2, psm2),
      (pmx3, psm3),
      cnts1,
      cnts2,
      Wl1,
      bl1.reshape(1, -1),
      Wl2,
      bl2.reshape(1, -1),
      Wl3,
      bl3.reshape(1, -1),
  )
